# baseline (speedup 1.0000x reference)
import functools

import jax
import jax.numpy as jnp
from jax import lax
from jax.experimental import pallas as pl
from jax.experimental.pallas import tpu as pltpu
from jax.experimental.pallas import tpu_sc as plsc


@functools.cache
def _make_broadcast_kernel(batch, seq_len, dim, dtype):
    info = plsc.get_sparse_core_info()
    num_workers = info.num_cores * info.num_subcores
    num_cores = info.num_cores
    assert seq_len % num_workers == 0
    rows_per_worker = seq_len // num_workers
    chunk = min(32, rows_per_worker)
    assert rows_per_worker % chunk == 0
    n_chunks = rows_per_worker // chunk

    mesh = plsc.VectorSubcoreMesh(core_axis_name="c", subcore_axis_name="s")

    @functools.partial(
        pl.kernel,
        out_type=jax.ShapeDtypeStruct((batch, seq_len, dim), dtype),
        mesh=mesh,
        scratch_types=[
            pltpu.VMEM((2, chunk, dim), dtype),
            pltpu.SemaphoreType.DMA,
            pltpu.SemaphoreType.DMA,
        ],
    )
    def bcast(w_hbm, out_hbm, bufs, rsem, wsem):
        wid = lax.axis_index("s") * num_cores + lax.axis_index("c")
        base = wid * rows_per_worker

        def start_read(c):
            r0 = base + c * chunk
            return pltpu.async_copy(
                w_hbm.at[pl.ds(r0, chunk)], bufs.at[c % 2], rsem
            )

        rh = [None] * n_chunks
        wh = [[] for _ in range(n_chunks)]
        rh[0] = start_read(0)
        for c in range(n_chunks):
            if c + 1 < n_chunks:
                for h in wh[c - 1] if c >= 1 else ():
                    h.wait()
                rh[c + 1] = start_read(c + 1)
            rh[c].wait()
            r0 = base + c * chunk
            for b in range(batch):
                wh[c].append(
                    pltpu.async_copy(
                        bufs.at[c % 2], out_hbm.at[b, pl.ds(r0, chunk)], wsem
                    )
                )
        for c in range(max(0, n_chunks - 2), n_chunks):
            for h in wh[c]:
                h.wait()

    return bcast


def kernel(x, weight):
    batch, seq_len, dim = x.shape
    return _make_broadcast_kernel(batch, seq_len, dim, weight.dtype)(weight)
